# baseline (speedup 1.0000x reference)
import jax
import jax.numpy as jnp
from jax import lax
from jax.experimental import pallas as pl
from jax.experimental.pallas import tpu as pltpu
from jax.experimental.pallas import tpu_sc as plsc

VOCAB = 1000000
EMSIZE = 64
PADE = 128
BATCH = 4096
SEQ = 200

_INFO = plsc.get_sparse_core_info()
_NC = _INFO.num_cores
_NS = _INFO.num_subcores
_NW = _NC * _NS
_ROWS = BATCH * SEQ
_RPW = _ROWS // _NW
_CHUNK = 128
_NCHUNK = _RPW // _CHUNK
_NBUF = 5
_PREF = 2


def _sc_body(seq_hbm, word_hbm, pos2_hbm, out_hbm,
             idx_v, pos_sh, bufs_v, gsem, ssem, psem):
    sid = lax.axis_index("s")
    wid = sid * _NC + lax.axis_index("c")

    pltpu.sync_copy(seq_hbm.at[wid], idx_v)

    @pl.when(sid == 0)
    def _():
        pltpu.sync_copy(pos2_hbm, pos_sh)

    plsc.subcore_barrier()

    base = wid * _RPW

    def prefill_start(c, b):
        off = (c * _CHUNK) % SEQ
        pltpu.async_copy(pos_sh.at[pl.ds(off, _CHUNK)], bufs_v.at[b],
                         psem.at[b])

    def prefill_wait(c, b):
        off = (c * _CHUNK) % SEQ
        pltpu.make_async_copy(pos_sh.at[pl.ds(off, _CHUNK)], bufs_v.at[b],
                              psem.at[b]).wait()

    def gather_start(c, b):
        pltpu.async_copy(word_hbm.at[idx_v.at[c]], bufs_v.at[b], gsem.at[b],
                         add=True)

    def gather_wait(c, b):
        pltpu.make_async_copy(
            word_hbm.at[idx_v.at[c]], bufs_v.at[b], gsem.at[b]).wait()

    def store_start(c, b):
        pltpu.async_copy(
            bufs_v.at[b], out_hbm.at[pl.ds(base + c * _CHUNK, _CHUNK)],
            ssem.at[b])

    def store_wait(c, b):
        pltpu.make_async_copy(
            bufs_v.at[b], out_hbm.at[pl.ds(base + c * _CHUNK, _CHUNK)],
            ssem.at[b]).wait()

    for p in range(_PREF):
        prefill_start(p, p)
    prefill_start(_PREF, _PREF % _NBUF)
    for p in range(_PREF):
        prefill_wait(p, p)
        gather_start(p, p)

    @pl.loop(0, _NCHUNK, step=_NBUF)
    def chunk_loop(c0):
        for j in range(_NBUF):
            c = c0 + j

            cp = c + _PREF + 1
            bp = (j + _PREF + 1) % _NBUF

            @pl.when(jnp.logical_and(cp >= _NBUF, cp < _NCHUNK))
            def _():
                store_wait(cp - _NBUF, bp)

            @pl.when(cp < _NCHUNK)
            def _():
                prefill_start(cp, bp)

            cn = c + _PREF
            bn = (j + _PREF) % _NBUF

            @pl.when(cn < _NCHUNK)
            def _():
                prefill_wait(cn, bn)
                gather_start(cn, bn)

            gather_wait(c, j)
            store_start(c, j)

    for j in range(_NBUF):
        store_wait(_NCHUNK - _NBUF + j, j)


@jax.jit
def _embed(seq_r, word128, pos2):
    mesh = plsc.VectorSubcoreMesh(core_axis_name="c", subcore_axis_name="s")
    k = pl.kernel(
        _sc_body,
        out_type=jax.ShapeDtypeStruct((_ROWS, PADE), jnp.float32),
        mesh=mesh,
        compiler_params=pltpu.CompilerParams(use_tc_tiling_on_sc=False),
        scratch_types=[
            pltpu.VMEM((_NCHUNK, _CHUNK), jnp.int32),
            pltpu.VMEM_SHARED((2 * SEQ, PADE), jnp.float32),
            pltpu.VMEM((_NBUF, _CHUNK, PADE), jnp.float32),
            pltpu.SemaphoreType.DMA((_NBUF,)),
            pltpu.SemaphoreType.DMA((_NBUF,)),
            pltpu.SemaphoreType.DMA((_NBUF,)),
        ],
    )
    return k(seq_r, word128, pos2)


def kernel(sequence, word_table, pos_table):
    seq_r = sequence.astype(jnp.int32).reshape(_NW, _NCHUNK, _CHUNK)
    word128 = jnp.pad(word_table, ((0, 0), (0, PADE - EMSIZE)))
    pos2 = jnp.pad(jnp.concatenate([pos_table, pos_table], axis=0),
                   ((0, 0), (0, PADE - EMSIZE)))
    out = _embed(seq_r, word128, pos2)
    return out[:, :EMSIZE].reshape(BATCH, SEQ, EMSIZE)


# speedup vs baseline: 1.0709x; 1.0709x over previous
import jax
import jax.numpy as jnp
from jax import lax
from jax.experimental import pallas as pl
from jax.experimental.pallas import tpu as pltpu
from jax.experimental.pallas import tpu_sc as plsc

VOCAB = 1000000
EMSIZE = 64
PADE = 128
BATCH = 4096
SEQ = 200

_INFO = plsc.get_sparse_core_info()
_NC = _INFO.num_cores
_NS = _INFO.num_subcores
_NW = _NC * _NS
_ROWS = BATCH * SEQ
_RPW = _ROWS // _NW
_CHUNK = 128
_NCHUNK = _RPW // _CHUNK
_NBUF = 5
_PREF = 2


def _sc_body(seq_hbm, word_hbm, pos2_hbm, out_hbm,
             idx_v, pos_sh, bufs_v, gsem, ssem, psem):
    sid = lax.axis_index("s")
    wid = sid * _NC + lax.axis_index("c")

    pltpu.sync_copy(seq_hbm.at[wid], idx_v)

    @pl.when(sid == 0)
    def _():
        pltpu.sync_copy(pos2_hbm, pos_sh)

    plsc.subcore_barrier()

    base = wid * _RPW

    def prefill_start(c, b):
        off = (c * _CHUNK) % SEQ
        pltpu.async_copy(pos_sh.at[pl.ds(off, _CHUNK)],
                         bufs_v.at[b].at[:, pl.ds(0, EMSIZE)], psem.at[b])

    def prefill_wait(c, b):
        off = (c * _CHUNK) % SEQ
        pltpu.make_async_copy(pos_sh.at[pl.ds(off, _CHUNK)],
                              bufs_v.at[b].at[:, pl.ds(0, EMSIZE)],
                              psem.at[b]).wait()

    def gather_start(c, b):
        pltpu.async_copy(word_hbm.at[idx_v.at[c]], bufs_v.at[b], gsem.at[b],
                         add=True)

    def gather_wait(c, b):
        pltpu.make_async_copy(
            word_hbm.at[idx_v.at[c]], bufs_v.at[b], gsem.at[b]).wait()

    def store_start(c, b):
        pltpu.async_copy(
            bufs_v.at[b].at[:, pl.ds(0, EMSIZE)],
            out_hbm.at[pl.ds(base + c * _CHUNK, _CHUNK), pl.ds(0, EMSIZE)],
            ssem.at[b])

    def store_wait(c, b):
        pltpu.make_async_copy(
            bufs_v.at[b].at[:, pl.ds(0, EMSIZE)],
            out_hbm.at[pl.ds(base + c * _CHUNK, _CHUNK), pl.ds(0, EMSIZE)],
            ssem.at[b]).wait()

    for p in range(_PREF):
        prefill_start(p, p)
    prefill_start(_PREF, _PREF % _NBUF)
    for p in range(_PREF):
        prefill_wait(p, p)
        gather_start(p, p)

    @pl.loop(0, _NCHUNK, step=_NBUF)
    def chunk_loop(c0):
        for j in range(_NBUF):
            c = c0 + j

            cp = c + _PREF + 1
            bp = (j + _PREF + 1) % _NBUF

            @pl.when(jnp.logical_and(cp >= _NBUF, cp < _NCHUNK))
            def _():
                store_wait(cp - _NBUF, bp)

            @pl.when(cp < _NCHUNK)
            def _():
                prefill_start(cp, bp)

            cn = c + _PREF
            bn = (j + _PREF) % _NBUF

            @pl.when(cn < _NCHUNK)
            def _():
                prefill_wait(cn, bn)
                gather_start(cn, bn)

            gather_wait(c, j)
            store_start(c, j)

    for j in range(_NBUF):
        store_wait(_NCHUNK - _NBUF + j, j)


@jax.jit
def _embed(seq_r, word128, pos2):
    mesh = plsc.VectorSubcoreMesh(core_axis_name="c", subcore_axis_name="s")
    k = pl.kernel(
        _sc_body,
        out_type=jax.ShapeDtypeStruct((_ROWS, PADE), jnp.float32),
        mesh=mesh,
        compiler_params=pltpu.CompilerParams(use_tc_tiling_on_sc=False),
        scratch_types=[
            pltpu.VMEM((_NCHUNK, _CHUNK), jnp.int32),
            pltpu.VMEM_SHARED((2 * SEQ, EMSIZE), jnp.float32),
            pltpu.VMEM((_NBUF, _CHUNK, PADE), jnp.float32),
            pltpu.SemaphoreType.DMA((_NBUF,)),
            pltpu.SemaphoreType.DMA((_NBUF,)),
            pltpu.SemaphoreType.DMA((_NBUF,)),
        ],
    )
    return k(seq_r, word128, pos2)


def kernel(sequence, word_table, pos_table):
    seq_r = sequence.astype(jnp.int32).reshape(_NW, _NCHUNK, _CHUNK)
    word128 = jnp.pad(word_table, ((0, 0), (0, PADE - EMSIZE)))
    pos2 = jnp.concatenate([pos_table, pos_table], axis=0)
    out = _embed(seq_r, word128, pos2)
    return out[:, :EMSIZE].reshape(BATCH, SEQ, EMSIZE)


# speedup vs baseline: 1.1595x; 1.0828x over previous
import jax
import jax.numpy as jnp
from jax import lax
from jax.experimental import pallas as pl
from jax.experimental.pallas import tpu as pltpu
from jax.experimental.pallas import tpu_sc as plsc

VOCAB = 1000000
EMSIZE = 64
BATCH = 4096
SEQ = 200

_INFO = plsc.get_sparse_core_info()
_NC = _INFO.num_cores
_NS = _INFO.num_subcores
_NW = _NC * _NS
_ROWS = BATCH * SEQ
_RPW = _ROWS // _NW
_CHUNK = 128
_NCHUNK = _RPW // _CHUNK
_NBUF = 8
_PREF = 4


def _sc_body(seq_hbm, word_hbm, pos2_hbm, out_hbm,
             idx_v, pos_v, pos_sh, bufs_v, gsem, ssem, psem):
    sid = lax.axis_index("s")
    wid = sid * _NC + lax.axis_index("c")

    pltpu.sync_copy(seq_hbm.at[wid], idx_v)
    pltpu.sync_copy(pos2_hbm, pos_v)

    @pl.when(sid == 0)
    def _():
        pltpu.sync_copy(pos_v, pos_sh)

    plsc.subcore_barrier()

    base = wid * _RPW

    def prefill_start(c, b):
        off = (c * _CHUNK) % SEQ
        pltpu.async_copy(pos_sh.at[pl.ds(off, _CHUNK)], bufs_v.at[b],
                         psem.at[b])

    def prefill_wait(c, b):
        off = (c * _CHUNK) % SEQ
        pltpu.make_async_copy(pos_sh.at[pl.ds(off, _CHUNK)], bufs_v.at[b],
                              psem.at[b]).wait()

    def gather_start(c, b):
        pltpu.async_copy(word_hbm.at[idx_v.at[c]], bufs_v.at[b], gsem.at[b],
                         add=True)

    def gather_wait(c, b):
        pltpu.make_async_copy(
            word_hbm.at[idx_v.at[c]], bufs_v.at[b], gsem.at[b]).wait()

    def store_start(c, b):
        pltpu.async_copy(
            bufs_v.at[b],
            out_hbm.at[pl.ds(base + c * _CHUNK, _CHUNK), pl.ds(0, EMSIZE)],
            ssem.at[b])

    def store_wait(c, b):
        pltpu.make_async_copy(
            bufs_v.at[b],
            out_hbm.at[pl.ds(base + c * _CHUNK, _CHUNK), pl.ds(0, EMSIZE)],
            ssem.at[b]).wait()

    for p in range(_PREF):
        prefill_start(p, p)
    prefill_start(_PREF, _PREF % _NBUF)
    for p in range(_PREF):
        prefill_wait(p, p)
        gather_start(p, p)

    @pl.loop(0, _NCHUNK, step=_NBUF)
    def chunk_loop(c0):
        for j in range(_NBUF):
            c = c0 + j

            cp = c + _PREF + 1
            bp = (j + _PREF + 1) % _NBUF

            @pl.when(jnp.logical_and(cp >= _NBUF, cp < _NCHUNK))
            def _():
                store_wait(cp - _NBUF, bp)

            @pl.when(cp < _NCHUNK)
            def _():
                prefill_start(cp, bp)

            cn = c + _PREF
            bn = (j + _PREF) % _NBUF

            @pl.when(cn < _NCHUNK)
            def _():
                prefill_wait(cn, bn)
                gather_start(cn, bn)

            gather_wait(c, j)
            store_start(c, j)

    for j in range(_NBUF):
        store_wait(_NCHUNK - _NBUF + j, j)


@jax.jit
def _embed(seq_r, word_table, pos2):
    mesh = plsc.VectorSubcoreMesh(core_axis_name="c", subcore_axis_name="s")
    k = pl.kernel(
        _sc_body,
        out_type=jax.ShapeDtypeStruct((_ROWS, 128), jnp.float32),
        mesh=mesh,
        compiler_params=pltpu.CompilerParams(use_tc_tiling_on_sc=False),
        scratch_types=[
            pltpu.VMEM((_NCHUNK, _CHUNK), jnp.int32),
            pltpu.VMEM((2 * SEQ, EMSIZE), jnp.float32),
            pltpu.VMEM_SHARED((2 * SEQ, EMSIZE), jnp.float32),
            pltpu.VMEM((_NBUF, _CHUNK, EMSIZE), jnp.float32),
            pltpu.SemaphoreType.DMA((_NBUF,)),
            pltpu.SemaphoreType.DMA((_NBUF,)),
            pltpu.SemaphoreType.DMA((_NBUF,)),
        ],
    )
    return k(seq_r, word_table, pos2)


def kernel(sequence, word_table, pos_table):
    seq_r = (2 * sequence.astype(jnp.int32)).reshape(_NW, _NCHUNK, _CHUNK)
    word2 = jnp.pad(word_table, ((0, 0), (0, EMSIZE))).reshape(
        2 * VOCAB, EMSIZE)
    pos2 = jnp.concatenate([pos_table, pos_table], axis=0)
    out = _embed(seq_r, word2, pos2)
    return out[:, :EMSIZE].reshape(BATCH, SEQ, EMSIZE)


# speedup vs baseline: 1.1698x; 1.0088x over previous
import jax
import jax.numpy as jnp
from jax import lax
from jax.experimental import pallas as pl
from jax.experimental.pallas import tpu as pltpu
from jax.experimental.pallas import tpu_sc as plsc

VOCAB = 1000000
EMSIZE = 64
BATCH = 4096
SEQ = 200

_INFO = plsc.get_sparse_core_info()
_NC = _INFO.num_cores
_NS = _INFO.num_subcores
_NW = _NC * _NS
_ROWS = BATCH * SEQ
_RPW = _ROWS // _NW
_CHUNK = 256
_NCHUNK = _RPW // _CHUNK
_NBUF = 5
_PREF = 2
_POSREP = 3


def _sc_body(seq_hbm, word_hbm, pos2_hbm, out_hbm,
             idx_v, pos_sh, bufs_v, gsem, ssem, psem):
    sid = lax.axis_index("s")
    wid = sid * _NC + lax.axis_index("c")

    pltpu.sync_copy(seq_hbm.at[wid], idx_v)

    @pl.when(sid == 0)
    def _():
        pltpu.sync_copy(pos2_hbm, pos_sh)

    plsc.subcore_barrier()

    base = wid * _RPW

    def prefill_start(c, b):
        off = (c * _CHUNK) % SEQ
        pltpu.async_copy(pos_sh.at[pl.ds(off, _CHUNK)], bufs_v.at[b],
                         psem.at[b])

    def prefill_wait(c, b):
        off = (c * _CHUNK) % SEQ
        pltpu.make_async_copy(pos_sh.at[pl.ds(off, _CHUNK)], bufs_v.at[b],
                              psem.at[b]).wait()

    def gather_start(c, b):
        pltpu.async_copy(word_hbm.at[idx_v.at[c]], bufs_v.at[b], gsem.at[b],
                         add=True)

    def gather_wait(c, b):
        pltpu.make_async_copy(
            word_hbm.at[idx_v.at[c]], bufs_v.at[b], gsem.at[b]).wait()

    def store_start(c, b):
        pltpu.async_copy(
            bufs_v.at[b],
            out_hbm.at[pl.ds(base + c * _CHUNK, _CHUNK), pl.ds(0, EMSIZE)],
            ssem.at[b])

    def store_wait(c, b):
        pltpu.make_async_copy(
            bufs_v.at[b],
            out_hbm.at[pl.ds(base + c * _CHUNK, _CHUNK), pl.ds(0, EMSIZE)],
            ssem.at[b]).wait()

    for p in range(_PREF):
        prefill_start(p, p)
    prefill_start(_PREF, _PREF % _NBUF)
    for p in range(_PREF):
        prefill_wait(p, p)
        gather_start(p, p)

    @pl.loop(0, _NCHUNK, step=_NBUF)
    def chunk_loop(c0):
        for j in range(_NBUF):
            c = c0 + j

            cp = c + _PREF + 1
            bp = (j + _PREF + 1) % _NBUF

            @pl.when(jnp.logical_and(cp >= _NBUF, cp < _NCHUNK))
            def _():
                store_wait(cp - _NBUF, bp)

            @pl.when(cp < _NCHUNK)
            def _():
                prefill_start(cp, bp)

            cn = c + _PREF
            bn = (j + _PREF) % _NBUF

            @pl.when(cn < _NCHUNK)
            def _():
                prefill_wait(cn, bn)
                gather_start(cn, bn)

            gather_wait(c, j)
            store_start(c, j)

    for j in range(_NBUF):
        store_wait(_NCHUNK - _NBUF + j, j)


@jax.jit
def _embed(seq_r, word_table, pos2):
    mesh = plsc.VectorSubcoreMesh(core_axis_name="c", subcore_axis_name="s")
    k = pl.kernel(
        _sc_body,
        out_type=jax.ShapeDtypeStruct((_ROWS, 128), jnp.float32),
        mesh=mesh,
        compiler_params=pltpu.CompilerParams(use_tc_tiling_on_sc=False),
        scratch_types=[
            pltpu.VMEM((_NCHUNK, _CHUNK), jnp.int32),
            pltpu.VMEM_SHARED((_POSREP * SEQ, EMSIZE), jnp.float32),
            pltpu.VMEM((_NBUF, _CHUNK, EMSIZE), jnp.float32),
            pltpu.SemaphoreType.DMA((_NBUF,)),
            pltpu.SemaphoreType.DMA((_NBUF,)),
            pltpu.SemaphoreType.DMA((_NBUF,)),
        ],
    )
    return k(seq_r, word_table, pos2)


def kernel(sequence, word_table, pos_table):
    seq_r = (2 * sequence.astype(jnp.int32)).reshape(_NW, _NCHUNK, _CHUNK)
    word2 = jnp.pad(word_table, ((0, 0), (0, EMSIZE))).reshape(
        2 * VOCAB, EMSIZE)
    pos2 = jnp.concatenate([pos_table] * _POSREP, axis=0)
    out = _embed(seq_r, word2, pos2)
    return out[:, :EMSIZE].reshape(BATCH, SEQ, EMSIZE)
